# Initial kernel scaffold; baseline (speedup 1.0000x reference)
import jax
import jax.numpy as jnp
from jax.experimental import pallas as pl


def kernel(x, edge_index, batch, W0, b0, W1, b1, W2, b2, proto_node_emb, last_w):
    raise NotImplementedError("write your pallas kernel here")


# speedup vs baseline: 9.0686x; 9.0686x over previous
import functools

import jax
import jax.numpy as jnp
from jax import lax
from jax.experimental import pallas as pl
from jax.experimental.pallas import tpu as pltpu
from jax.experimental.pallas import tpu_sc as plsc

NC = 2
NS = 16
NW = NC * NS
CHUNK = 128
D = 128
NG = 128
BM = 1024
EPS = 1e-4


def _deg_call(dst_p, n_pad, e_pad):
    rows_per_tile = n_pad // NS
    ew = e_pad // NW
    steps = ew // CHUNK
    mesh = plsc.VectorSubcoreMesh(core_axis_name="c", subcore_axis_name="s")

    @functools.partial(
        pl.kernel,
        out_type=jax.ShapeDtypeStruct((NC, n_pad, 16), jnp.float32),
        mesh=mesh,
        scratch_types=[
            pltpu.VMEM((CHUNK,), jnp.int32),
            pltpu.VMEM((CHUNK, 16), jnp.float32),
            pltpu.VMEM((CHUNK, 16), jnp.float32),
            pltpu.VMEM_SHARED((n_pad, 16), jnp.float32),
        ],
    )
    def k(dst_hbm, out_hbm, idx_v, ones_v, zb_v, acc):
        c = lax.axis_index("c")
        s = lax.axis_index("s")
        w = s * NC + c

        def fill(r, _):
            ones_v[r, :] = jnp.ones((16,), jnp.float32)
            zb_v[r, :] = jnp.zeros((16,), jnp.float32)
            return 0
        lax.fori_loop(0, CHUNK, fill, 0)
        for j in range(rows_per_tile // CHUNK):
            pltpu.sync_copy(zb_v, acc.at[pl.ds(s * rows_per_tile + j * CHUNK, CHUNK)])
        plsc.subcore_barrier()

        base = w * ew

        def step(i, _):
            pltpu.sync_copy(dst_hbm.at[pl.ds(base + i * CHUNK, CHUNK)], idx_v)
            pltpu.sync_copy(ones_v, acc.at[idx_v], add=True)
            return 0
        lax.fori_loop(0, steps, step, 0)
        plsc.subcore_barrier()
        pltpu.sync_copy(acc.at[pl.ds(s * rows_per_tile, rows_per_tile)],
                        out_hbm.at[c, pl.ds(s * rows_per_tile, rows_per_tile)])

    return k(dst_p)


def _agg_call(g, src_p, dst_p, n_pad, e_pad):
    rows_per_tile = n_pad // NS
    ew = e_pad // NW
    steps = ew // CHUNK
    mesh = plsc.VectorSubcoreMesh(core_axis_name="c", subcore_axis_name="s")

    @functools.partial(
        pl.kernel,
        out_type=jax.ShapeDtypeStruct((NC, n_pad, D), jnp.float32),
        mesh=mesh,
        scratch_types=[
            pltpu.VMEM((CHUNK,), jnp.int32),
            pltpu.VMEM((CHUNK,), jnp.int32),
            pltpu.VMEM((CHUNK, D), jnp.float32),
            pltpu.VMEM_SHARED((n_pad, D), jnp.float32),
            pltpu.SemaphoreType.DMA,
        ],
    )
    def k(g_hbm, src_hbm, dst_hbm, out_hbm, sidx, didx, rows, acc, sem):
        c = lax.axis_index("c")
        s = lax.axis_index("s")
        w = s * NC + c

        def fillz(r, _):
            for j in range(D // 16):
                rows[r, pl.ds(j * 16, 16)] = jnp.zeros((16,), jnp.float32)
            return 0
        lax.fori_loop(0, CHUNK, fillz, 0)
        for j in range(rows_per_tile // CHUNK):
            pltpu.sync_copy(rows, acc.at[pl.ds(s * rows_per_tile + j * CHUNK, CHUNK)])
        plsc.subcore_barrier()

        base = w * ew

        def step(i, _):
            pltpu.sync_copy(src_hbm.at[pl.ds(base + i * CHUNK, CHUNK)], sidx)
            pltpu.sync_copy(dst_hbm.at[pl.ds(base + i * CHUNK, CHUNK)], didx)
            pltpu.async_copy(g_hbm.at[sidx], rows, sem).wait()
            pltpu.sync_copy(rows, acc.at[didx], add=True)
            return 0
        lax.fori_loop(0, steps, step, 0)
        plsc.subcore_barrier()
        pltpu.sync_copy(acc.at[pl.ds(s * rows_per_tile, rows_per_tile)],
                        out_hbm.at[c, pl.ds(s * rows_per_tile, rows_per_tile)])

    return k(g, src_p, dst_p)


def _dinv_of(dp):
    return lax.rsqrt(1.0 + dp[0, :, :1] + dp[1, :, :1])


def _prep_body(dp_ref, x_ref, w_ref, t_ref, g_ref):
    dinv = _dinv_of(dp_ref[...])
    t = jnp.dot(x_ref[...], w_ref[...], preferred_element_type=jnp.float32)
    t_ref[...] = t
    g_ref[...] = t * dinv


def _layer_body(p_ref, t_ref, dp_ref, b_ref, w_ref, tn_ref, gn_ref):
    dinv = _dinv_of(dp_ref[...])
    p = p_ref[...]
    h = jnp.maximum(dinv * (p[0] + p[1]) + (dinv * dinv) * t_ref[...] + b_ref[...], 0.0)
    t = jnp.dot(h, w_ref[...], preferred_element_type=jnp.float32)
    tn_ref[...] = t
    gn_ref[...] = t * dinv


def _readout_body(p_ref, t_ref, dp_ref, b_ref, batch_ref, proto_ref, lw_ref,
                  out_ref, acc_s, acc_c):
    i = pl.program_id(0)

    @pl.when(i == 0)
    def _():
        acc_s[...] = jnp.zeros_like(acc_s)
        acc_c[...] = jnp.zeros_like(acc_c)

    dinv = _dinv_of(dp_ref[...])
    p = p_ref[...]
    h = jnp.maximum(dinv * (p[0] + p[1]) + (dinv * dinv) * t_ref[...] + b_ref[...], 0.0)
    b_ids = batch_ref[...].reshape(1, BM)
    gids = lax.broadcasted_iota(jnp.int32, (NG, BM), 0)
    onehot_t = jnp.where(gids == b_ids, 1.0, 0.0)
    acc_s[...] += jnp.dot(onehot_t, h, preferred_element_type=jnp.float32)
    acc_c[...] += jnp.sum(onehot_t, axis=1, keepdims=True)

    @pl.when(i == pl.num_programs(0) - 1)
    def _():
        cnt = jnp.maximum(acc_c[...], 1.0)
        emb = acc_s[...] / cnt
        proto = proto_ref[...]
        gsz = proto.shape[1]
        pg = proto[:, 0, :]
        for j in range(1, gsz):
            pg = pg + proto[:, j, :]
        pg = pg * (1.0 / gsz)
        cross = lax.dot_general(emb, pg, (((1,), (1,)), ((), ())),
                                preferred_element_type=jnp.float32)
        d2 = (jnp.sum(emb * emb, axis=1, keepdims=True)
              + jnp.sum(pg * pg, axis=1)[None, :]
              - 2.0 * cross)
        d2 = jnp.maximum(d2, 0.0)
        sim = jnp.log((d2 + 1.0) / (d2 + EPS))
        out_ref[...] = lax.dot_general(sim, lw_ref[...], (((1,), (1,)), ((), ())),
                                       preferred_element_type=jnp.float32)


def kernel(x, edge_index, batch, W0, b0, W1, b1, W2, b2, proto_node_emb, last_w):
    n, d = x.shape
    n_pad = ((n + BM - 1) // BM) * BM
    e = edge_index.shape[1]
    e_pad = ((e + NW * CHUNK - 1) // (NW * CHUNK)) * (NW * CHUNK)

    src_p = jnp.concatenate(
        [edge_index[0].astype(jnp.int32),
         jnp.full((e_pad - e,), n_pad - 1, jnp.int32)])
    dst_p = jnp.concatenate(
        [edge_index[1].astype(jnp.int32),
         jnp.full((e_pad - e,), n_pad - 1, jnp.int32)])
    x_p = jnp.pad(x, ((0, n_pad - n), (0, 0)))
    batch3 = jnp.pad(batch.astype(jnp.int32), (0, n_pad - n),
                     constant_values=NG).reshape(n_pad // BM, 1, BM)
    b0r, b1r, b2r = (b.reshape(1, d) for b in (b0, b1, b2))

    nblk = n_pad // BM
    degparts = _deg_call(dst_p, n_pad, e_pad)

    dp_spec = pl.BlockSpec((2, BM, 16), lambda i: (0, i, 0))
    row_spec = pl.BlockSpec((BM, d), lambda i: (i, 0))
    mat_spec = pl.BlockSpec((d, d), lambda i: (0, 0))
    p_spec = pl.BlockSpec((2, BM, d), lambda i: (0, i, 0))
    td2 = [jax.ShapeDtypeStruct((n_pad, d), jnp.float32)] * 2

    t0, g0 = pl.pallas_call(
        _prep_body, grid=(nblk,),
        in_specs=[dp_spec, row_spec, mat_spec],
        out_specs=[row_spec, row_spec], out_shape=td2,
    )(degparts, x_p, W0)

    p = _agg_call(g0, src_p, dst_p, n_pad, e_pad)

    bias_spec = pl.BlockSpec((1, d), lambda i: (0, 0))
    layer = pl.pallas_call(
        _layer_body, grid=(nblk,),
        in_specs=[p_spec, row_spec, dp_spec, bias_spec, mat_spec],
        out_specs=[row_spec, row_spec], out_shape=td2,
    )
    t1, g1 = layer(p, t0, degparts, b0r, W1)
    p = _agg_call(g1, src_p, dst_p, n_pad, e_pad)
    t2, g2 = layer(p, t1, degparts, b1r, W2)
    p = _agg_call(g2, src_p, dst_p, n_pad, e_pad)

    np_, gsz, _ = proto_node_emb.shape
    c = last_w.shape[0]
    logits = pl.pallas_call(
        _readout_body, grid=(nblk,),
        in_specs=[p_spec, row_spec, dp_spec, bias_spec,
                  pl.BlockSpec((1, 1, BM), lambda i: (i, 0, 0)),
                  pl.BlockSpec((np_, gsz, d), lambda i: (0, 0, 0)),
                  pl.BlockSpec((c, np_), lambda i: (0, 0))],
        out_specs=pl.BlockSpec((NG, c), lambda i: (0, 0)),
        out_shape=jax.ShapeDtypeStruct((NG, c), jnp.float32),
        scratch_shapes=[pltpu.VMEM((NG, d), jnp.float32),
                        pltpu.VMEM((NG, 1), jnp.float32)],
    )(p, t2, degparts, b2r, batch3, proto_node_emb, last_w)
    return logits
